# baseline (speedup 1.0000x reference)
import functools

import numpy as np
import jax
import jax.numpy as jnp
from jax import lax
from jax.experimental import pallas as pl
from jax.experimental.pallas import tpu as pltpu
from jax.experimental.pallas import tpu_sc as plsc

_N = 16384
_D = 2048
_R = 745
_DWI = 128
_DWB = 2 * _DWI
_NQ = 4
_TQ = _N // _NQ
_CT = 128
_CO = 64
_NCHUNK = _TQ // _CT
_L = 16


def _build_combine_matrix() -> np.ndarray:
    m = np.zeros((_R, 197), np.float32)
    r = 0
    for s in range(60):
        m[r, 84 + s] = 1.0
        r += 1
    for mi in range(60):
        for y in range(3):
            m[r, 24 + mi] = 1.0
            m[r, 187 + y] = 1.0
            r += 1
    for d in range(31):
        for w in range(7):
            m[r, 144 + d] = 1.0
            m[r, 190 + w] = 1.0
            r += 1
    for h in range(24):
        for mo in range(12):
            m[r, 0 + h] = 1.0
            m[r, 175 + mo] = 1.0
            r += 1
    assert r == _R
    return m


_M_COMBINE = _build_combine_matrix()


def _sum_lookup_tables(m, refs):
    f32 = jnp.float32
    offs = [(0, 24), (24, 84), (84, 144), (144, 175), (175, 187),
            (187, 190), (190, 197)]
    t = None
    for (lo, hi), r in zip(offs, refs):
        d = jnp.dot(m[:, lo:hi], r[...], preferred_element_type=f32)
        t = d if t is None else t + d
    return t


def _round_to_bf16_bits(x):
    b = lax.bitcast_convert_type(x, jnp.int32)
    lsb = lax.shift_right_logical(b, 16) & 1
    return lax.shift_right_logical(b + 0x7FFF + lsb, 16)


def _combine_body(m_ref, *refs):
    lo_refs = refs[0:7]
    hi_refs = refs[7:14]
    o_ref = refs[14]
    m = m_ref[...]
    tlo = _sum_lookup_tables(m, lo_refs)
    thi = _sum_lookup_tables(m, hi_refs)
    o_ref[...] = (
        lax.shift_left(_round_to_bf16_bits(thi), 16)
        | _round_to_bf16_bits(tlo))


def _combine_tables(m, tables):
    db = 512
    ng = _D // 2 // db
    rows = [24, 60, 60, 31, 12, 3, 7]

    def lospec(r):
        return pl.BlockSpec((r, db), lambda i: (0, i))

    def hispec(r):
        return pl.BlockSpec((r, db), lambda i: (0, i + ng))

    return pl.pallas_call(
        _combine_body,
        grid=(ng,),
        in_specs=([pl.BlockSpec((_R, 197), lambda i: (0, 0))]
                  + [lospec(r) for r in rows] + [hispec(r) for r in rows]),
        out_specs=pl.BlockSpec((_R, db), lambda i: (0, i)),
        out_shape=jax.ShapeDtypeStruct((_R, _D // 2), jnp.int32),
    )(m, *tables, *tables)


def _sc_body(t_hbm, tf_hbm, out_hbm, table_v, tf_v0, tf_v1, out_v0, out_v1,
             semt0, semt1, semo0, semo1):
    cid = lax.axis_index("c")
    sid = lax.axis_index("s")
    cc = lax.rem(sid, 8)
    tq = cid * 2 + lax.div(sid, 8)
    wcol0 = cc * _DWI
    lcol0 = cc * _DWI
    hcol0 = _D // 2 + cc * _DWI
    tokb = tq * _TQ

    pltpu.sync_copy(t_hbm.at[:, pl.ds(wcol0, _DWI)], table_v)

    def compute_sub(tok0, tf_vb, tf_off, out_vb, sem_out):
        @plsc.parallel_loop(0, _CO, step=_L, unroll=2)
        def _group(base):
            hh = tf_vb[0, pl.ds(tf_off + base, _L)]
            mi = tf_vb[1, pl.ds(tf_off + base, _L)]
            se = tf_vb[2, pl.ds(tf_off + base, _L)]
            dy = tf_vb[3, pl.ds(tf_off + base, _L)]
            mo = tf_vb[4, pl.ds(tf_off + base, _L)]
            yr = tf_vb[5, pl.ds(tf_off + base, _L)]
            wd = tf_vb[6, pl.ds(tf_off + base, _L)]
            i0v = se
            i1v = mi * 3 + yr - 1949
            i2v = dy * 7 + wd + 233
            i3v = hh * 12 + mo + 456
            for l in range(_L):
                a = i0v[l]
                b = i1v[l]
                c = i2v[l]
                d = i3v[l]
                halves = []
                for cg in range(_DWI // _L):
                    sl = pl.ds(cg * _L, _L)
                    acc = ((plsc.bitcast(table_v[a, sl], jnp.bfloat16)
                            + plsc.bitcast(table_v[b, sl], jnp.bfloat16))
                           + (plsc.bitcast(table_v[c, sl], jnp.bfloat16)
                              + plsc.bitcast(table_v[d, sl], jnp.bfloat16)))
                    halves.append(plsc.unpack(
                        acc, format=plsc.PackFormat.INTERLEAVED))
                for cg in range(_DWI // _L):
                    lo, hi = halves[cg]
                    out_vb[base + l, pl.ds(cg * _L, _L)] = lo
                    out_vb[base + l, pl.ds(_DWI + cg * _L, _L)] = hi

        pltpu.async_copy(
            out_vb.at[:, pl.ds(0, _DWI)],
            out_hbm.at[pl.ds(tok0, _CO), pl.ds(lcol0, _DWI)], sem_out)
        pltpu.async_copy(
            out_vb.at[:, pl.ds(_DWI, _DWI)],
            out_hbm.at[pl.ds(tok0, _CO), pl.ds(hcol0, _DWI)], sem_out)

    def wait_tf(tf_vb, semt):
        pltpu.make_async_copy(
            tf_hbm.at[:, pl.ds(0, _CT)], tf_vb, semt).wait()

    def fetch_tf(k, tf_vb, semt):
        pltpu.async_copy(
            tf_hbm.at[:, pl.ds(tokb + k * _CT, _CT)], tf_vb, semt)

    def wait_out(out_vb, semo):
        for _ in range(2):
            pltpu.make_async_copy(
                out_vb.at[:, pl.ds(0, _DWI)],
                out_hbm.at[pl.ds(0, _CO), pl.ds(0, _DWI)], semo).wait()

    def chunk(k, tf_vb, first):
        tok0 = tokb + k * _CT

        @pl.when(jnp.logical_not(first))
        def _():
            wait_out(out_v0, semo0)

        compute_sub(tok0, tf_vb, 0, out_v0, semo0)

        @pl.when(jnp.logical_not(first))
        def _():
            wait_out(out_v1, semo1)

        compute_sub(tok0 + _CO, tf_vb, _CO, out_v1, semo1)

    fetch_tf(0, tf_v0, semt0)

    def chunk_pair(p, _):
        k0 = p * 2

        wait_tf(tf_v0, semt0)
        fetch_tf(k0 + 1, tf_v1, semt1)
        chunk(k0, tf_v0, p == 0)

        wait_tf(tf_v1, semt1)

        @pl.when(p < _NCHUNK // 2 - 1)
        def _():
            fetch_tf(k0 + 2, tf_v0, semt0)

        chunk(k0 + 1, tf_v1, False)
        return 0

    lax.fori_loop(0, _NCHUNK // 2, chunk_pair, 0)
    wait_out(out_v0, semo0)
    wait_out(out_v1, semo1)


def _sc_lookup(table_i32, tf_t):
    mesh = plsc.VectorSubcoreMesh(core_axis_name="c", subcore_axis_name="s")
    run = functools.partial(
        pl.kernel,
        mesh=mesh,
        compiler_params=pltpu.CompilerParams(needs_layout_passes=False),
        out_type=jax.ShapeDtypeStruct((_N, _D), jnp.float32),
        scratch_types=[
            pltpu.VMEM((_R, _DWI), jnp.int32),
            pltpu.VMEM((7, _CT), jnp.int32),
            pltpu.VMEM((7, _CT), jnp.int32),
            pltpu.VMEM((_CO, _DWB), jnp.float32),
            pltpu.VMEM((_CO, _DWB), jnp.float32),
            pltpu.SemaphoreType.DMA,
            pltpu.SemaphoreType.DMA,
            pltpu.SemaphoreType.DMA,
            pltpu.SemaphoreType.DMA,
        ],
    )(_sc_body)
    return run(table_i32, tf_t)


def kernel(time_features, hour_w, minute_w, second_w, day_w, month_w,
           year_w, weekday_w):
    tables = [hour_w, minute_w, second_w, day_w, month_w, year_w,
              weekday_w]
    table_i32 = _combine_tables(jnp.asarray(_M_COMBINE), tables)
    tf_t = time_features.T
    return _sc_lookup(table_i32, tf_t)


# speedup vs baseline: 2.3690x; 2.3690x over previous
import functools

import numpy as np
import jax
import jax.numpy as jnp
from jax import lax
from jax.experimental import pallas as pl
from jax.experimental.pallas import tpu as pltpu
from jax.experimental.pallas import tpu_sc as plsc

_N = 16384
_D = 2048
_R = 745
_DWI = 128
_DWB = 2 * _DWI
_NQ = 4
_TQ = _N // _NQ
_CT = 128
_CO = 64
_NCHUNK = _TQ // _CT
_L = 16


def _build_combine_matrix() -> np.ndarray:
    m = np.zeros((_R, 197), np.float32)
    r = 0
    for s in range(60):
        m[r, 84 + s] = 1.0
        r += 1
    for mi in range(60):
        for y in range(3):
            m[r, 24 + mi] = 1.0
            m[r, 187 + y] = 1.0
            r += 1
    for d in range(31):
        for w in range(7):
            m[r, 144 + d] = 1.0
            m[r, 190 + w] = 1.0
            r += 1
    for h in range(24):
        for mo in range(12):
            m[r, 0 + h] = 1.0
            m[r, 175 + mo] = 1.0
            r += 1
    assert r == _R
    return m


_M_COMBINE = _build_combine_matrix()


def _sum_lookup_tables(m, refs):
    f32 = jnp.float32
    offs = [(0, 24), (24, 84), (84, 144), (144, 175), (175, 187),
            (187, 190), (190, 197)]
    t = None
    for (lo, hi), r in zip(offs, refs):
        d = jnp.dot(m[:, lo:hi], r[...], preferred_element_type=f32)
        t = d if t is None else t + d
    return t


def _round_to_bf16_bits(x):
    b = lax.bitcast_convert_type(x, jnp.int32)
    lsb = lax.shift_right_logical(b, 16) & 1
    return lax.shift_right_logical(b + 0x7FFF + lsb, 16)


def _combine_body(m_ref, *refs):
    lo_refs = refs[0:7]
    hi_refs = refs[7:14]
    o_ref = refs[14]
    m = m_ref[...]
    tlo = _sum_lookup_tables(m, lo_refs)
    thi = _sum_lookup_tables(m, hi_refs)
    o_ref[...] = (
        lax.shift_left(_round_to_bf16_bits(thi), 16)
        | _round_to_bf16_bits(tlo))


def _combine_tables(m, tables):
    db = 512
    ng = _D // 2 // db
    rows = [24, 60, 60, 31, 12, 3, 7]

    def lospec(r):
        return pl.BlockSpec((r, db), lambda i: (0, i))

    def hispec(r):
        return pl.BlockSpec((r, db), lambda i: (0, i + ng))

    return pl.pallas_call(
        _combine_body,
        grid=(ng,),
        in_specs=([pl.BlockSpec((_R, 197), lambda i: (0, 0))]
                  + [lospec(r) for r in rows] + [hispec(r) for r in rows]),
        out_specs=pl.BlockSpec((_R, db), lambda i: (0, i)),
        out_shape=jax.ShapeDtypeStruct((_R, _D // 2), jnp.int32),
    )(m, *tables, *tables)


def _sc_body(t_hbm, tf_hbm, out_hbm, table_v, tf_v0, tf_v1, out_v0, out_v1,
             semt0, semt1, semo0, semo1):
    cid = lax.axis_index("c")
    sid = lax.axis_index("s")
    cc = lax.rem(sid, 8)
    tq = cid * 2 + lax.div(sid, 8)
    wcol0 = cc * _DWI
    lcol0 = cc * _DWI
    hcol0 = _D // 2 + cc * _DWI
    tokb = tq * _TQ

    pltpu.sync_copy(t_hbm.at[:, pl.ds(wcol0, _DWI)], table_v)

    def compute_sub(tok0, tf_vb, tf_off, out_vb, sem_out):
        @plsc.parallel_loop(0, _CO, step=_L)
        def _group(base):
            hh = tf_vb[0, pl.ds(tf_off + base, _L)]
            mi = tf_vb[1, pl.ds(tf_off + base, _L)]
            se = tf_vb[2, pl.ds(tf_off + base, _L)]
            dy = tf_vb[3, pl.ds(tf_off + base, _L)]
            mo = tf_vb[4, pl.ds(tf_off + base, _L)]
            yr = tf_vb[5, pl.ds(tf_off + base, _L)]
            wd = tf_vb[6, pl.ds(tf_off + base, _L)]
            i0v = se
            i1v = mi * 3 + yr - 1949
            i2v = dy * 7 + wd + 233
            i3v = hh * 12 + mo + 456
            for l in range(_L):
                a = i0v[l]
                b = i1v[l]
                c = i2v[l]
                d = i3v[l]
                halves = []
                for cg in range(_DWI // _L):
                    sl = pl.ds(cg * _L, _L)
                    acc = ((plsc.bitcast(table_v[a, sl], jnp.bfloat16)
                            + plsc.bitcast(table_v[b, sl], jnp.bfloat16))
                           + (plsc.bitcast(table_v[c, sl], jnp.bfloat16)
                              + plsc.bitcast(table_v[d, sl], jnp.bfloat16)))
                    halves.append(plsc.unpack(
                        acc, format=plsc.PackFormat.INTERLEAVED))
                for cg in range(_DWI // _L):
                    lo, hi = halves[cg]
                    out_vb[base + l, pl.ds(cg * _L, _L)] = lo
                    out_vb[base + l, pl.ds(_DWI + cg * _L, _L)] = hi

        pltpu.async_copy(
            out_vb.at[:, pl.ds(0, _DWI)],
            out_hbm.at[pl.ds(tok0, _CO), pl.ds(lcol0, _DWI)], sem_out)
        pltpu.async_copy(
            out_vb.at[:, pl.ds(_DWI, _DWI)],
            out_hbm.at[pl.ds(tok0, _CO), pl.ds(hcol0, _DWI)], sem_out)

    def wait_tf(tf_vb, semt):
        pltpu.make_async_copy(
            tf_hbm.at[:, pl.ds(0, _CT)], tf_vb, semt).wait()

    def fetch_tf(k, tf_vb, semt):
        pltpu.async_copy(
            tf_hbm.at[:, pl.ds(tokb + k * _CT, _CT)], tf_vb, semt)

    def wait_out(out_vb, semo):
        for _ in range(2):
            pltpu.make_async_copy(
                out_vb.at[:, pl.ds(0, _DWI)],
                out_hbm.at[pl.ds(0, _CO), pl.ds(0, _DWI)], semo).wait()

    def chunk(k, tf_vb, first):
        tok0 = tokb + k * _CT

        @pl.when(jnp.logical_not(first))
        def _():
            wait_out(out_v0, semo0)

        compute_sub(tok0, tf_vb, 0, out_v0, semo0)

        @pl.when(jnp.logical_not(first))
        def _():
            wait_out(out_v1, semo1)

        compute_sub(tok0 + _CO, tf_vb, _CO, out_v1, semo1)

    fetch_tf(0, tf_v0, semt0)

    def chunk_pair(p, _):
        k0 = p * 2

        wait_tf(tf_v0, semt0)
        fetch_tf(k0 + 1, tf_v1, semt1)
        chunk(k0, tf_v0, p == 0)

        wait_tf(tf_v1, semt1)

        @pl.when(p < _NCHUNK // 2 - 1)
        def _():
            fetch_tf(k0 + 2, tf_v0, semt0)

        chunk(k0 + 1, tf_v1, False)
        return 0

    lax.fori_loop(0, _NCHUNK // 2, chunk_pair, 0)
    wait_out(out_v0, semo0)
    wait_out(out_v1, semo1)


def _sc_lookup(table_i32, tf_t):
    mesh = plsc.VectorSubcoreMesh(core_axis_name="c", subcore_axis_name="s")
    run = functools.partial(
        pl.kernel,
        mesh=mesh,
        compiler_params=pltpu.CompilerParams(needs_layout_passes=False),
        out_type=jax.ShapeDtypeStruct((_N, _D), jnp.float32),
        scratch_types=[
            pltpu.VMEM((_R, _DWI), jnp.int32),
            pltpu.VMEM((7, _CT), jnp.int32),
            pltpu.VMEM((7, _CT), jnp.int32),
            pltpu.VMEM((_CO, _DWB), jnp.float32),
            pltpu.VMEM((_CO, _DWB), jnp.float32),
            pltpu.SemaphoreType.DMA,
            pltpu.SemaphoreType.DMA,
            pltpu.SemaphoreType.DMA,
            pltpu.SemaphoreType.DMA,
        ],
    )(_sc_body)
    return run(table_i32, tf_t)


def kernel(time_features, hour_w, minute_w, second_w, day_w, month_w,
           year_w, weekday_w):
    tables = [hour_w, minute_w, second_w, day_w, month_w, year_w,
              weekday_w]
    table_i32 = _combine_tables(jnp.asarray(_M_COMBINE), tables)
    tf_t = time_features.T
    return _sc_lookup(table_i32, tf_t)
